# baseline (speedup 1.0000x reference)
import jax
import jax.numpy as jnp
from jax.experimental import pallas as pl
from jax.experimental.pallas import tpu as pltpu

B, P, D = 4096, 8192, 64
BLOCK_B = 512


def _centroid_kernel(x_ref, c_ref, mask_ref, out_ref, cn_ref, cext_ref):
    @pl.when(pl.program_id(0) == 0)
    def _prep():
        c = c_ref[...]
        m = mask_ref[...].reshape(P, 1)
        cn = c / (jnp.sqrt(jnp.sum(c * c, axis=-1, keepdims=True)) + 1e-12)
        cn_ref[...] = cn.astype(jnp.bfloat16)
        cext_ref[...] = jnp.concatenate([c * m, m], axis=1)

    x = x_ref[...]
    xn = x / (jnp.sqrt(jnp.sum(x * x, axis=-1, keepdims=True)) + 1e-12)

    sim = jax.lax.dot_general(
        xn.astype(jnp.bfloat16), cn_ref[...], (((1,), (1,)), ((), ())),
        preferred_element_type=jnp.float32)
    e = jnp.exp(sim)

    ctx = jax.lax.dot_general(
        e, cext_ref[...], (((1,), (0,)), ((), ())),
        preferred_element_type=jnp.float32)
    out_ref[...] = ctx[:, :D] / ctx[:, D:D + 1]


@jax.jit
def kernel(x, centroid_emb, active_mask):
    maskf = active_mask.astype(jnp.float32).reshape(1, P)
    return pl.pallas_call(
        _centroid_kernel,
        grid=(B // BLOCK_B,),
        in_specs=[
            pl.BlockSpec((BLOCK_B, D), lambda i: (i, 0)),
            pl.BlockSpec((P, D), lambda i: (0, 0)),
            pl.BlockSpec((1, P), lambda i: (0, 0)),
        ],
        out_specs=pl.BlockSpec((BLOCK_B, D), lambda i: (i, 0)),
        out_shape=jax.ShapeDtypeStruct((B, D), jnp.float32),
        scratch_shapes=[
            pltpu.VMEM((P, D), jnp.bfloat16),
            pltpu.VMEM((P, D + 1), jnp.float32),
        ],
    )(x, centroid_emb, maskf)


# speedup vs baseline: 1.0366x; 1.0366x over previous
import jax
import jax.numpy as jnp
from jax.experimental import pallas as pl
from jax.experimental.pallas import tpu as pltpu

B, P, D = 4096, 8192, 64
BLOCK_B = 1024


def _centroid_kernel(x_ref, c_ref, mask_ref, out_ref, cn_ref, cm_ref, bias_ref):
    @pl.when(pl.program_id(0) == 0)
    def _prep():
        c = c_ref[...]
        m = mask_ref[...]
        cn = c / (jnp.sqrt(jnp.sum(c * c, axis=-1, keepdims=True)) + 1e-12)
        cn_ref[...] = cn.astype(jnp.bfloat16)
        cm_ref[...] = (c * m.reshape(P, 1)).astype(jnp.bfloat16)
        bias_ref[...] = jnp.where(m > 0, 0.0, -1e9).astype(jnp.float32)

    x = x_ref[...]
    xn = x / (jnp.sqrt(jnp.sum(x * x, axis=-1, keepdims=True)) + 1e-12)

    sim = jax.lax.dot_general(
        xn.astype(jnp.bfloat16), cn_ref[...], (((1,), (1,)), ((), ())),
        preferred_element_type=jnp.float32)
    e = jnp.exp(sim + bias_ref[...])
    s = jnp.sum(e, axis=-1, keepdims=True)

    ctx = jax.lax.dot_general(
        e, cm_ref[...], (((1,), (0,)), ((), ())),
        preferred_element_type=jnp.float32)
    out_ref[...] = ctx / s


@jax.jit
def kernel(x, centroid_emb, active_mask):
    maskf = active_mask.astype(jnp.float32).reshape(1, P)
    return pl.pallas_call(
        _centroid_kernel,
        grid=(B // BLOCK_B,),
        in_specs=[
            pl.BlockSpec((BLOCK_B, D), lambda i: (i, 0)),
            pl.BlockSpec((P, D), lambda i: (0, 0)),
            pl.BlockSpec((1, P), lambda i: (0, 0)),
        ],
        out_specs=pl.BlockSpec((BLOCK_B, D), lambda i: (i, 0)),
        out_shape=jax.ShapeDtypeStruct((B, D), jnp.float32),
        scratch_shapes=[
            pltpu.VMEM((P, D), jnp.bfloat16),
            pltpu.VMEM((P, D), jnp.bfloat16),
            pltpu.VMEM((1, P), jnp.float32),
        ],
    )(x, centroid_emb, maskf)
